# initial kernel scaffold (unmeasured)
import jax
import jax.numpy as jnp
from jax import lax
from jax.experimental import pallas as pl
from jax.experimental.pallas import tpu as pltpu

N_DEV = 8


def kernel(x, w_mat, scale_x, scale_w):
    k_full, k_loc = x.shape
    _, n = w_mat.shape
    m_loc = k_full // N_DEV
    print(f"[kernel trace] x={x.shape}/{x.dtype} w={w_mat.shape}/{w_mat.dtype} "
          f"sx={scale_x.shape}/{scale_x.dtype}")

    def body(x_ref, w_ref, sx_ref, sw_ref, out_ref, xg_ref, send_sems, recv_sems):
        i = lax.axis_index("i")

        xg_ref[:, pl.ds(i * k_loc, k_loc)] = x_ref[pl.ds(i * m_loc, m_loc), :]

        sends = []
        for s in range(1, N_DEV):
            dst = lax.rem(i + s, N_DEV)
            rdma = pltpu.make_async_remote_copy(
                src_ref=x_ref.at[pl.ds(dst * m_loc, m_loc), :],
                dst_ref=xg_ref.at[:, pl.ds(i * k_loc, k_loc)],
                send_sem=send_sems.at[s - 1],
                recv_sem=recv_sems.at[s - 1],
                device_id=(dst,),
                device_id_type=pl.DeviceIdType.MESH,
            )
            rdma.start()
            sends.append(rdma)

        for s in range(1, N_DEV):
            src = lax.rem(i - s + N_DEV, N_DEV)
            recv = pltpu.make_async_remote_copy(
                src_ref=x_ref.at[pl.ds(0, m_loc), :],
                dst_ref=xg_ref.at[:, pl.ds(src * k_loc, k_loc)],
                send_sem=send_sems.at[s - 1],
                recv_sem=recv_sems.at[s - 1],
                device_id=(i,),
                device_id_type=pl.DeviceIdType.MESH,
            )
            recv.wait_recv()

        acc = lax.dot_general(
            xg_ref[:, :], w_ref[:, :],
            (((1,), (0,)), ((), ())),
            preferred_element_type=jnp.float32,
        )
        y = acc * (sx_ref[0] * sw_ref[0])
        out_ref[:, :] = y * jax.nn.sigmoid(y)

        for rdma in sends:
            rdma.wait_send()

    return pl.pallas_call(
        body,
        out_shape=jax.ShapeDtypeStruct((m_loc, n), jnp.float32),
        in_specs=[
            pl.BlockSpec(memory_space=pltpu.VMEM),
            pl.BlockSpec(memory_space=pltpu.VMEM),
            pl.BlockSpec(memory_space=pltpu.SMEM),
            pl.BlockSpec(memory_space=pltpu.SMEM),
        ],
        out_specs=pl.BlockSpec(memory_space=pltpu.VMEM),
        scratch_shapes=[
            pltpu.VMEM((m_loc, k_full), x.dtype),
            pltpu.SemaphoreType.DMA((N_DEV - 1,)),
            pltpu.SemaphoreType.DMA((N_DEV - 1,)),
        ],
    )(x, w_mat, scale_x, scale_w)


# baseline (device time: 92194 ns/iter reference)
import jax
import jax.numpy as jnp
from jax import lax
from jax.experimental import pallas as pl
from jax.experimental.pallas import tpu as pltpu

N_DEV = 8
NT = 512


def kernel(x, w_mat, scale_x, scale_w):
    k_full, k_loc = x.shape
    _, n = w_mat.shape
    m_loc = k_full // N_DEV
    n_tiles = n // NT

    def body(x_ref, w_hbm, sx_ref, sw_ref, out_ref,
             xf8_ref, xg_ref, wtile_ref, copy_sems, send_sems, recv_sems):
        i = lax.axis_index("i")

        for t in range(2):
            pltpu.make_async_copy(
                w_hbm.at[:, pl.ds(t * NT, NT)], wtile_ref.at[t], copy_sems.at[t]
            ).start()

        xf8_ref[:, :] = x_ref[:, :].astype(jnp.float8_e4m3fn)

        xg_ref[:, pl.ds(i * k_loc, k_loc)] = xf8_ref[pl.ds(i * m_loc, m_loc), :]

        sends = []
        for s in range(1, N_DEV):
            dst = lax.rem(i + s, N_DEV)
            rdma = pltpu.make_async_remote_copy(
                src_ref=xf8_ref.at[pl.ds(dst * m_loc, m_loc), :],
                dst_ref=xg_ref.at[:, pl.ds(i * k_loc, k_loc)],
                send_sem=send_sems.at[s - 1],
                recv_sem=recv_sems.at[s - 1],
                device_id=(dst,),
                device_id_type=pl.DeviceIdType.MESH,
            )
            rdma.start()
            sends.append(rdma)

        for s in range(1, N_DEV):
            src = lax.rem(i - s + N_DEV, N_DEV)
            recv = pltpu.make_async_remote_copy(
                src_ref=xf8_ref.at[pl.ds(0, m_loc), :],
                dst_ref=xg_ref.at[:, pl.ds(src * k_loc, k_loc)],
                send_sem=send_sems.at[s - 1],
                recv_sem=recv_sems.at[s - 1],
                device_id=(i,),
                device_id_type=pl.DeviceIdType.MESH,
            )
            recv.wait_recv()

        xg_bf = xg_ref[:, :].astype(jnp.bfloat16)
        scale = sx_ref[0] * sw_ref[0]

        def gemm_step(t, carry):
            slot = lax.rem(t, 2)
            pltpu.make_async_copy(
                w_hbm.at[:, pl.ds(t * NT, NT)], wtile_ref.at[slot],
                copy_sems.at[slot],
            ).wait()
            wt_bf = wtile_ref[slot, :, :].astype(jnp.bfloat16)
            acc = lax.dot_general(
                xg_bf, wt_bf, (((1,), (0,)), ((), ())),
                preferred_element_type=jnp.float32,
            )
            y = acc * scale
            out_ref[:, pl.ds(t * NT, NT)] = y * jax.nn.sigmoid(y)

            @pl.when(t + 2 < n_tiles)
            def _():
                pltpu.make_async_copy(
                    w_hbm.at[:, pl.ds((t + 2) * NT, NT)], wtile_ref.at[slot],
                    copy_sems.at[slot],
                ).start()

            return carry

        lax.fori_loop(0, n_tiles, gemm_step, 0)

        for rdma in sends:
            rdma.wait_send()

    return pl.pallas_call(
        body,
        out_shape=jax.ShapeDtypeStruct((m_loc, n), jnp.float32),
        in_specs=[
            pl.BlockSpec(memory_space=pltpu.VMEM),
            pl.BlockSpec(memory_space=pl.ANY),
            pl.BlockSpec(memory_space=pltpu.SMEM),
            pl.BlockSpec(memory_space=pltpu.SMEM),
        ],
        out_specs=pl.BlockSpec(memory_space=pltpu.VMEM),
        scratch_shapes=[
            pltpu.VMEM((k_full, k_loc), jnp.float8_e4m3fn),
            pltpu.VMEM((m_loc, k_full), jnp.float8_e4m3fn),
            pltpu.VMEM((2, k_full, NT), jnp.float32),
            pltpu.SemaphoreType.DMA((2,)),
            pltpu.SemaphoreType.DMA((N_DEV - 1,)),
            pltpu.SemaphoreType.DMA((N_DEV - 1,)),
        ],
        compiler_params=pltpu.CompilerParams(
            vmem_limit_bytes=64 * 1024 * 1024,
        ),
    )(x, w_mat, scale_x, scale_w)


# device time: 49732 ns/iter; 1.8538x vs baseline; 1.8538x over previous
import jax
import jax.numpy as jnp
from jax import lax
from jax.experimental import pallas as pl
from jax.experimental.pallas import tpu as pltpu

import os

N_DEV = 8
NT = 512
_SKIP_A2A = os.environ.get("KERNEL_SKIP_A2A", "0") == "1"
_SKIP_GEMM = os.environ.get("KERNEL_SKIP_GEMM", "0") == "1"


def kernel(x, w_mat, scale_x, scale_w):
    k_full, k_loc = x.shape
    _, n = w_mat.shape
    m_loc = k_full // N_DEV
    n_tiles = n // NT

    def body(x_ref, w_hbm, sx_ref, sw_ref, out_ref,
             xf8_ref, xg_ref, wtile_ref, copy_sems, send_sems, recv_sems):
        i = lax.axis_index("i")

        for t in range(2):
            pltpu.make_async_copy(
                w_hbm.at[:, pl.ds(t * NT, NT)], wtile_ref.at[t], copy_sems.at[t]
            ).start()

        xf8_ref[:, :] = x_ref[:, :].astype(jnp.float8_e4m3fn)

        xg_ref[:, pl.ds(i * k_loc, k_loc)] = xf8_ref[pl.ds(i * m_loc, m_loc), :]

        sends = []
        for s in range(1, N_DEV) if not _SKIP_A2A else []:
            dst = lax.rem(i + s, N_DEV)
            rdma = pltpu.make_async_remote_copy(
                src_ref=xf8_ref.at[pl.ds(dst * m_loc, m_loc), :],
                dst_ref=xg_ref.at[:, pl.ds(i * k_loc, k_loc)],
                send_sem=send_sems.at[s - 1],
                recv_sem=recv_sems.at[s - 1],
                device_id=(dst,),
                device_id_type=pl.DeviceIdType.MESH,
            )
            rdma.start()
            sends.append(rdma)

        for s in range(1, N_DEV) if not _SKIP_A2A else []:
            src = lax.rem(i - s + N_DEV, N_DEV)
            recv = pltpu.make_async_remote_copy(
                src_ref=xf8_ref.at[pl.ds(0, m_loc), :],
                dst_ref=xg_ref.at[:, pl.ds(src * k_loc, k_loc)],
                send_sem=send_sems.at[s - 1],
                recv_sem=recv_sems.at[s - 1],
                device_id=(i,),
                device_id_type=pl.DeviceIdType.MESH,
            )
            recv.wait_recv()

        xg_bf = xg_ref[:, :].astype(jnp.bfloat16)
        scale = sx_ref[0] * sw_ref[0]

        def gemm_step(t, carry):
            slot = lax.rem(t, 2)
            pltpu.make_async_copy(
                w_hbm.at[:, pl.ds(t * NT, NT)], wtile_ref.at[slot],
                copy_sems.at[slot],
            ).wait()
            wt_bf = wtile_ref[slot, :, :].astype(jnp.bfloat16)
            acc = lax.dot_general(
                xg_bf, wt_bf, (((1,), (0,)), ((), ())),
                preferred_element_type=jnp.float32,
            )
            y = acc * scale
            out_ref[:, pl.ds(t * NT, NT)] = y * jax.nn.sigmoid(y)

            @pl.when(t + 2 < n_tiles)
            def _():
                pltpu.make_async_copy(
                    w_hbm.at[:, pl.ds((t + 2) * NT, NT)], wtile_ref.at[slot],
                    copy_sems.at[slot],
                ).start()

            return carry

        def dma_only_step(t, carry):
            slot = lax.rem(t, 2)
            pltpu.make_async_copy(
                w_hbm.at[:, pl.ds(t * NT, NT)], wtile_ref.at[slot],
                copy_sems.at[slot],
            ).wait()
            out_ref[:, pl.ds(t * NT, NT)] = jnp.zeros((m_loc, NT), jnp.float32)

            @pl.when(t + 2 < n_tiles)
            def _():
                pltpu.make_async_copy(
                    w_hbm.at[:, pl.ds((t + 2) * NT, NT)], wtile_ref.at[slot],
                    copy_sems.at[slot],
                ).start()

            return carry

        lax.fori_loop(0, n_tiles, dma_only_step if _SKIP_GEMM else gemm_step, 0)

        for rdma in sends:
            rdma.wait_send()

    return pl.pallas_call(
        body,
        out_shape=jax.ShapeDtypeStruct((m_loc, n), jnp.float32),
        in_specs=[
            pl.BlockSpec(memory_space=pltpu.VMEM),
            pl.BlockSpec(memory_space=pl.ANY),
            pl.BlockSpec(memory_space=pltpu.SMEM),
            pl.BlockSpec(memory_space=pltpu.SMEM),
        ],
        out_specs=pl.BlockSpec(memory_space=pltpu.VMEM),
        scratch_shapes=[
            pltpu.VMEM((k_full, k_loc), jnp.float8_e4m3fn),
            pltpu.VMEM((m_loc, k_full), jnp.float8_e4m3fn),
            pltpu.VMEM((2, k_full, NT), jnp.float32),
            pltpu.SemaphoreType.DMA((2,)),
            pltpu.SemaphoreType.DMA((N_DEV - 1,)),
            pltpu.SemaphoreType.DMA((N_DEV - 1,)),
        ],
        compiler_params=pltpu.CompilerParams(
            vmem_limit_bytes=64 * 1024 * 1024,
        ),
    )(x, w_mat, scale_x, scale_w)


# device time: 46328 ns/iter; 1.9900x vs baseline; 1.0735x over previous
import jax
import jax.numpy as jnp
from jax import lax
from jax.experimental import pallas as pl
from jax.experimental.pallas import tpu as pltpu

import os

N_DEV = 8
NT = 512
_SKIP_A2A = os.environ.get("KERNEL_SKIP_A2A", "0") == "1"
_SKIP_GEMM = os.environ.get("KERNEL_SKIP_GEMM", "0") == "1"
_A2A_ONLY = os.environ.get("KERNEL_A2A_ONLY", "0") == "1"


def kernel(x, w_mat, scale_x, scale_w):
    k_full, k_loc = x.shape
    _, n = w_mat.shape
    m_loc = k_full // N_DEV
    n_tiles = n // NT

    def body(x_ref, w_hbm, sx_ref, sw_ref, out_ref,
             xf8_ref, xg_ref, wtile_ref, copy_sems, send_sems, recv_sems):
        i = lax.axis_index("i")

        for t in range(2) if not _A2A_ONLY else []:
            pltpu.make_async_copy(
                w_hbm.at[:, pl.ds(t * NT, NT)], wtile_ref.at[t], copy_sems.at[t]
            ).start()

        xf8_ref[:, :] = x_ref[:, :].astype(jnp.float8_e4m3fn)

        xg_ref[:, pl.ds(i * k_loc, k_loc)] = xf8_ref[pl.ds(i * m_loc, m_loc), :]

        sends = []
        for s in range(1, N_DEV) if not _SKIP_A2A else []:
            dst = lax.rem(i + s, N_DEV)
            rdma = pltpu.make_async_remote_copy(
                src_ref=xf8_ref.at[pl.ds(dst * m_loc, m_loc), :],
                dst_ref=xg_ref.at[:, pl.ds(i * k_loc, k_loc)],
                send_sem=send_sems.at[s - 1],
                recv_sem=recv_sems.at[s - 1],
                device_id=(dst,),
                device_id_type=pl.DeviceIdType.MESH,
            )
            rdma.start()
            sends.append(rdma)

        for s in range(1, N_DEV) if not _SKIP_A2A else []:
            src = lax.rem(i - s + N_DEV, N_DEV)
            recv = pltpu.make_async_remote_copy(
                src_ref=xf8_ref.at[pl.ds(0, m_loc), :],
                dst_ref=xg_ref.at[:, pl.ds(src * k_loc, k_loc)],
                send_sem=send_sems.at[s - 1],
                recv_sem=recv_sems.at[s - 1],
                device_id=(i,),
                device_id_type=pl.DeviceIdType.MESH,
            )
            recv.wait_recv()

        xg_bf = xg_ref[:, :].astype(jnp.bfloat16)
        scale = sx_ref[0] * sw_ref[0]

        def gemm_step(t, carry):
            slot = lax.rem(t, 2)
            pltpu.make_async_copy(
                w_hbm.at[:, pl.ds(t * NT, NT)], wtile_ref.at[slot],
                copy_sems.at[slot],
            ).wait()
            wt_bf = wtile_ref[slot, :, :].astype(jnp.bfloat16)
            acc = lax.dot_general(
                xg_bf, wt_bf, (((1,), (0,)), ((), ())),
                preferred_element_type=jnp.float32,
            )
            y = acc * scale
            out_ref[:, pl.ds(t * NT, NT)] = y * jax.nn.sigmoid(y)

            @pl.when(t + 2 < n_tiles)
            def _():
                pltpu.make_async_copy(
                    w_hbm.at[:, pl.ds((t + 2) * NT, NT)], wtile_ref.at[slot],
                    copy_sems.at[slot],
                ).start()

            return carry

        def dma_only_step(t, carry):
            slot = lax.rem(t, 2)
            pltpu.make_async_copy(
                w_hbm.at[:, pl.ds(t * NT, NT)], wtile_ref.at[slot],
                copy_sems.at[slot],
            ).wait()
            out_ref[:, pl.ds(t * NT, NT)] = jnp.zeros((m_loc, NT), jnp.float32)

            @pl.when(t + 2 < n_tiles)
            def _():
                pltpu.make_async_copy(
                    w_hbm.at[:, pl.ds((t + 2) * NT, NT)], wtile_ref.at[slot],
                    copy_sems.at[slot],
                ).start()

            return carry

        if _A2A_ONLY:
            out_ref[:, :] = jnp.zeros((m_loc, n), jnp.float32)
            out_ref[:, pl.ds(0, k_full)] = xg_ref[:, :].astype(jnp.float32)
        else:
            lax.fori_loop(0, n_tiles, dma_only_step if _SKIP_GEMM else gemm_step, 0)

        for rdma in sends:
            rdma.wait_send()

    return pl.pallas_call(
        body,
        out_shape=jax.ShapeDtypeStruct((m_loc, n), jnp.float32),
        in_specs=[
            pl.BlockSpec(memory_space=pltpu.VMEM),
            pl.BlockSpec(memory_space=pl.ANY),
            pl.BlockSpec(memory_space=pltpu.SMEM),
            pl.BlockSpec(memory_space=pltpu.SMEM),
        ],
        out_specs=pl.BlockSpec(memory_space=pltpu.VMEM),
        scratch_shapes=[
            pltpu.VMEM((k_full, k_loc), jnp.float8_e4m3fn),
            pltpu.VMEM((m_loc, k_full), jnp.float8_e4m3fn),
            pltpu.VMEM((2, k_full, NT), jnp.float32),
            pltpu.SemaphoreType.DMA((2,)),
            pltpu.SemaphoreType.DMA((N_DEV - 1,)),
            pltpu.SemaphoreType.DMA((N_DEV - 1,)),
        ],
        compiler_params=pltpu.CompilerParams(
            vmem_limit_bytes=64 * 1024 * 1024,
        ),
    )(x, w_mat, scale_x, scale_w)


# device time: 28291 ns/iter; 3.2588x vs baseline; 1.6376x over previous
import jax
import jax.numpy as jnp
from jax import lax
from jax.experimental import pallas as pl
from jax.experimental.pallas import tpu as pltpu

import os

N_DEV = 8
NT = 512
_SKIP_A2A = os.environ.get("KERNEL_SKIP_A2A", "0") == "1"
_SKIP_GEMM = os.environ.get("KERNEL_SKIP_GEMM", "0") == "1"
_A2A_ONLY = os.environ.get("KERNEL_A2A_ONLY", "0") == "1"
_BARRIER_ONLY = os.environ.get("KERNEL_BARRIER_ONLY", "0") == "1"


def kernel(x, w_mat, scale_x, scale_w):
    k_full, k_loc = x.shape
    _, n = w_mat.shape
    m_loc = k_full // N_DEV
    n_tiles = n // NT

    def body(x_ref, w_hbm, sx_ref, sw_ref, out_ref,
             xf8_ref, xg_ref, wtile_ref, copy_sems, send_sems, recv_sems):
        i = lax.axis_index("i")

        barrier_sem = pltpu.get_barrier_semaphore()
        for s in range(1, N_DEV):
            pl.semaphore_signal(
                barrier_sem, inc=1,
                device_id=(lax.rem(i + s, N_DEV),),
                device_id_type=pl.DeviceIdType.MESH,
            )
        pl.semaphore_wait(barrier_sem, N_DEV - 1)

        for t in range(2) if not _A2A_ONLY else []:
            pltpu.make_async_copy(
                w_hbm.at[:, pl.ds(t * NT, NT)], wtile_ref.at[t], copy_sems.at[t]
            ).start()

        xf8_ref[:, :] = x_ref[:, :].astype(jnp.float8_e4m3fn)

        xg_ref[:, pl.ds(i * k_loc, k_loc)] = xf8_ref[pl.ds(i * m_loc, m_loc), :]

        sends = []
        for s in range(1, N_DEV) if not (_SKIP_A2A or _BARRIER_ONLY) else []:
            dst = lax.rem(i + s, N_DEV)
            rdma = pltpu.make_async_remote_copy(
                src_ref=xf8_ref.at[pl.ds(dst * m_loc, m_loc), :],
                dst_ref=xg_ref.at[:, pl.ds(i * k_loc, k_loc)],
                send_sem=send_sems.at[s - 1],
                recv_sem=recv_sems.at[s - 1],
                device_id=(dst,),
                device_id_type=pl.DeviceIdType.MESH,
            )
            rdma.start()
            sends.append(rdma)

        for s in range(1, N_DEV) if not (_SKIP_A2A or _BARRIER_ONLY) else []:
            src = lax.rem(i - s + N_DEV, N_DEV)
            recv = pltpu.make_async_remote_copy(
                src_ref=xf8_ref.at[pl.ds(0, m_loc), :],
                dst_ref=xg_ref.at[:, pl.ds(src * k_loc, k_loc)],
                send_sem=send_sems.at[s - 1],
                recv_sem=recv_sems.at[s - 1],
                device_id=(i,),
                device_id_type=pl.DeviceIdType.MESH,
            )
            recv.wait_recv()

        xg_bf = xg_ref[:, :].astype(jnp.bfloat16)
        scale = sx_ref[0] * sw_ref[0]

        def gemm_step(t, carry):
            slot = lax.rem(t, 2)
            pltpu.make_async_copy(
                w_hbm.at[:, pl.ds(t * NT, NT)], wtile_ref.at[slot],
                copy_sems.at[slot],
            ).wait()
            wt_bf = wtile_ref[slot, :, :].astype(jnp.bfloat16)
            acc = lax.dot_general(
                xg_bf, wt_bf, (((1,), (0,)), ((), ())),
                preferred_element_type=jnp.float32,
            )
            y = acc * scale
            out_ref[:, pl.ds(t * NT, NT)] = y * jax.nn.sigmoid(y)

            @pl.when(t + 2 < n_tiles)
            def _():
                pltpu.make_async_copy(
                    w_hbm.at[:, pl.ds((t + 2) * NT, NT)], wtile_ref.at[slot],
                    copy_sems.at[slot],
                ).start()

            return carry

        def dma_only_step(t, carry):
            slot = lax.rem(t, 2)
            pltpu.make_async_copy(
                w_hbm.at[:, pl.ds(t * NT, NT)], wtile_ref.at[slot],
                copy_sems.at[slot],
            ).wait()
            out_ref[:, pl.ds(t * NT, NT)] = jnp.zeros((m_loc, NT), jnp.float32)

            @pl.when(t + 2 < n_tiles)
            def _():
                pltpu.make_async_copy(
                    w_hbm.at[:, pl.ds((t + 2) * NT, NT)], wtile_ref.at[slot],
                    copy_sems.at[slot],
                ).start()

            return carry

        if _A2A_ONLY:
            out_ref[:, :] = jnp.zeros((m_loc, n), jnp.float32)
            out_ref[:, pl.ds(0, k_full)] = xg_ref[:, :].astype(jnp.float32)
        else:
            lax.fori_loop(0, n_tiles, dma_only_step if _SKIP_GEMM else gemm_step, 0)

        for rdma in sends:
            rdma.wait_send()

    return pl.pallas_call(
        body,
        out_shape=jax.ShapeDtypeStruct((m_loc, n), jnp.float32),
        in_specs=[
            pl.BlockSpec(memory_space=pltpu.VMEM),
            pl.BlockSpec(memory_space=pl.ANY),
            pl.BlockSpec(memory_space=pltpu.SMEM),
            pl.BlockSpec(memory_space=pltpu.SMEM),
        ],
        out_specs=pl.BlockSpec(memory_space=pltpu.VMEM),
        scratch_shapes=[
            pltpu.VMEM((k_full, k_loc), jnp.float8_e4m3fn),
            pltpu.VMEM((m_loc, k_full), jnp.float8_e4m3fn),
            pltpu.VMEM((2, k_full, NT), jnp.float32),
            pltpu.SemaphoreType.DMA((2,)),
            pltpu.SemaphoreType.DMA((N_DEV - 1,)),
            pltpu.SemaphoreType.DMA((N_DEV - 1,)),
        ],
        compiler_params=pltpu.CompilerParams(
            vmem_limit_bytes=64 * 1024 * 1024,
            collective_id=0,
        ),
    )(x, w_mat, scale_x, scale_w)
